# baseline (speedup 1.0000x reference)
import functools

import jax
import jax.numpy as jnp
from jax import lax
from jax.experimental import pallas as pl
from jax.experimental.pallas import tpu as pltpu
from jax.experimental.pallas import tpu_sc as plsc

N_NODES = 100000
N_EDGES = 3200000
NC = 2
NS = 16
NW = NC * NS
E_PER_W = N_EDGES // NW
CHUNK = 4000
N_CHUNKS = E_PER_W // CHUNK
GROUPS = CHUNK // 16
N_PAD = 784 * 128
NPS = N_PAD // NS
W_GROUPS = NPS // 16
RB = 112
TC_GRID = 784 // RB

_mesh = plsc.VectorSubcoreMesh(core_axis_name="c", subcore_axis_name="s")
_sc_params = pltpu.CompilerParams(needs_layout_passes=False)


def _rsqrt16(b2, iters=2):
    i = plsc.bitcast(b2, jnp.int32)
    i = jnp.int32(0x5F3759DF) - (i >> 1)
    y = plsc.bitcast(i, jnp.float32)
    for _ in range(iters):
        y = y * (1.5 - 0.5 * b2 * y * y)
    return y


def _edge_geom(rx_buf, ry_buf, rz_buf, gi):
    gsl = pl.ds(gi * 16, 16)
    x = rx_buf[gsl]
    y = ry_buf[gsl]
    z = rz_buf[gsl]
    b2 = jnp.maximum(x * x + y * y + z * z, 1e-30)
    rinv = _rsqrt16(b2)
    return x, y, z, b2 * rinv, rinv


@functools.partial(
    pl.kernel,
    out_type=jax.ShapeDtypeStruct((NC, N_PAD), jnp.float32),
    mesh=_mesh,
    compiler_params=_sc_params,
    scratch_types=[
        pltpu.VMEM_SHARED((N_PAD,), jnp.float32),
        pltpu.VMEM((16,), jnp.float32),
    ] + [pltpu.VMEM((CHUNK,), jnp.float32)] * 3
      + [pltpu.VMEM((CHUNK,), jnp.int32)] * 2
      + [pltpu.VMEM((CHUNK,), jnp.float32)] * 2
      + [pltpu.SemaphoreType.DMA] * 2,
)
def _sc_density(rx_hbm, ry_hbm, rz_hbm, dst_hbm, par_hbm, zeros_hbm, rho_out,
                rho_sh, par_v,
                rx_buf, ry_buf, rz_buf, dsta, dstb, da, db,
                ssc_a, ssc_b):
    cid = lax.axis_index("c")
    sid = lax.axis_index("s")
    wid = sid * NC + cid
    sl = pl.ds(sid * NPS, NPS)
    pltpu.sync_copy(zeros_hbm.at[sl], rho_sh.at[sl])
    pltpu.sync_copy(par_hbm, par_v)
    plsc.subcore_barrier()

    pv = par_v[...]
    a_d = pv[0]
    l_d = pv[1]
    e_base = wid * E_PER_W

    def do_chunk(e0, dst_buf, d_buf, sem):
        pltpu.sync_copy(rx_hbm.at[pl.ds(e0, CHUNK)], rx_buf)
        pltpu.sync_copy(ry_hbm.at[pl.ds(e0, CHUNK)], ry_buf)
        pltpu.sync_copy(rz_hbm.at[pl.ds(e0, CHUNK)], rz_buf)
        pltpu.sync_copy(dst_hbm.at[pl.ds(e0, CHUNK)], dst_buf)

        def grp(gi, carry):
            _, _, _, b, _ = _edge_geom(rx_buf, ry_buf, rz_buf, gi)
            d_buf[pl.ds(gi * 16, 16)] = a_d * jnp.exp(-l_d * b)
            return carry

        lax.fori_loop(0, GROUPS, grp, 0, unroll=4)
        return pltpu.async_copy(d_buf, rho_sh.at[dst_buf], sem, add=True)

    def pair_body(ci, carry):
        e0 = e_base + ci * (2 * CHUNK)
        sc_a = do_chunk(e0, dsta, da, ssc_a)
        sc_b = do_chunk(e0 + CHUNK, dstb, db, ssc_b)
        sc_a.wait()
        sc_b.wait()
        return carry

    lax.fori_loop(0, (N_CHUNKS - 1) // 2, pair_body, 0)
    do_chunk(e_base + (N_CHUNKS - 1) * CHUNK, dsta, da, ssc_a).wait()
    plsc.subcore_barrier()
    pltpu.sync_copy(rho_sh.at[sl], rho_out.at[cid, sl])


@functools.partial(
    pl.kernel,
    out_type=(jax.ShapeDtypeStruct((NC * 3, N_PAD), jnp.float32),
              jax.ShapeDtypeStruct((NW, 16), jnp.float32),
              jax.ShapeDtypeStruct((NW, 16), jnp.float32)),
    mesh=_mesh,
    compiler_params=_sc_params,
    scratch_types=[
        pltpu.VMEM_SHARED((N_PAD,), jnp.float32),
        pltpu.VMEM_SHARED((N_PAD,), jnp.float32),
        pltpu.VMEM_SHARED((N_PAD,), jnp.float32),
        pltpu.VMEM_SHARED((N_PAD,), jnp.float32),
        pltpu.VMEM((NPS,), jnp.float32),
        pltpu.VMEM((NPS,), jnp.float32),
        pltpu.VMEM((16,), jnp.float32),
        pltpu.VMEM((16,), jnp.float32),
        pltpu.VMEM((16,), jnp.float32),
    ] + [pltpu.VMEM((CHUNK,), jnp.float32)] * 3
      + [pltpu.VMEM((CHUNK,), jnp.int32)] * 2
      + [pltpu.VMEM((CHUNK,), jnp.float32)] * 7
      + [pltpu.SemaphoreType.DMA],
)
def _sc_forces(rx_hbm, ry_hbm, rz_hbm, src_hbm, dst_hbm, rho_hbm, par_hbm,
               zeros_hbm, f_out, rep_out, sq_out,
               w_sh, fx_sh, fy_sh, fz_sh, p0_v, p1_v,
               par_v, acc_buf, sq_buf,
               rx_buf, ry_buf, rz_buf, src_buf, dst_buf,
               wv_buf, pfx, pfy, pfz, nfx, nfy, nfz,
               sem):
    cid = lax.axis_index("c")
    sid = lax.axis_index("s")
    wid = sid * NC + cid
    sl = pl.ds(sid * NPS, NPS)
    pltpu.sync_copy(rho_hbm.at[0, sl], p0_v)
    pltpu.sync_copy(rho_hbm.at[1, sl], p1_v)
    pltpu.sync_copy(zeros_hbm.at[sl], fx_sh.at[sl])
    pltpu.sync_copy(zeros_hbm.at[sl], fy_sh.at[sl])
    pltpu.sync_copy(zeros_hbm.at[sl], fz_sh.at[sl])
    pltpu.sync_copy(par_hbm, par_v)

    def wgrp(gi, acc):
        gsl = pl.ds(gi * 16, 16)
        rho = p0_v[gsl] + p1_v[gsl]
        wv = _rsqrt16(rho, iters=3)
        p0_v[gsl] = wv
        return acc + rho * wv

    sq = lax.fori_loop(0, W_GROUPS, wgrp, jnp.zeros((16,), jnp.float32))
    sq_buf[...] = sq
    pltpu.sync_copy(sq_buf, sq_out.at[wid])
    pltpu.sync_copy(p0_v, w_sh.at[sl])
    plsc.subcore_barrier()

    pv = par_v[...]
    a_d = pv[0]
    l_d = pv[1]
    a_r = pv[2]
    l_r = pv[3]
    k1 = pv[4]
    e_base = wid * E_PER_W

    def chunk_body(ci, acc):
        e0 = e_base + ci * CHUNK
        pltpu.sync_copy(rx_hbm.at[pl.ds(e0, CHUNK)], rx_buf)
        pltpu.sync_copy(ry_hbm.at[pl.ds(e0, CHUNK)], ry_buf)
        pltpu.sync_copy(rz_hbm.at[pl.ds(e0, CHUNK)], rz_buf)
        pltpu.sync_copy(src_hbm.at[pl.ds(e0, CHUNK)], src_buf)
        pltpu.sync_copy(dst_hbm.at[pl.ds(e0, CHUNK)], dst_buf)
        pltpu.async_copy(w_sh.at[dst_buf], wv_buf, sem).wait()

        def grp(gi, acc):
            x, y, z, b, rinv = _edge_geom(rx_buf, ry_buf, rz_buf, gi)
            gsl = pl.ds(gi * 16, 16)
            wv = wv_buf[gsl]
            d = a_d * jnp.exp(-l_d * b)
            rep = a_r * jnp.exp(-l_r * b)
            s = (l_r * rep - k1 * d * wv) * rinv
            sx = s * x
            sy = s * y
            sz = s * z
            pfx[gsl] = sx
            pfy[gsl] = sy
            pfz[gsl] = sz
            nfx[gsl] = -sx
            nfy[gsl] = -sy
            nfz[gsl] = -sz
            return acc + rep

        acc = lax.fori_loop(0, GROUPS, grp, acc)
        pltpu.sync_copy(pfx, fx_sh.at[dst_buf], add=True)
        pltpu.sync_copy(pfy, fy_sh.at[dst_buf], add=True)
        pltpu.sync_copy(pfz, fz_sh.at[dst_buf], add=True)
        pltpu.sync_copy(nfx, fx_sh.at[src_buf], add=True)
        pltpu.sync_copy(nfy, fy_sh.at[src_buf], add=True)
        pltpu.sync_copy(nfz, fz_sh.at[src_buf], add=True)
        return acc

    acc = lax.fori_loop(0, N_CHUNKS, chunk_body, jnp.zeros((16,), jnp.float32))
    acc_buf[...] = acc
    pltpu.sync_copy(acc_buf, rep_out.at[wid])
    plsc.subcore_barrier()
    pltpu.sync_copy(fx_sh.at[sl], f_out.at[cid * 3, sl])
    pltpu.sync_copy(fy_sh.at[sl], f_out.at[cid * 3 + 1, sl])
    pltpu.sync_copy(fz_sh.at[sl], f_out.at[cid * 3 + 2, sl])


def _tc_final_body(f_ref, rep_ref, sq_ref, ae_ref, fo_ref, en_ref):
    i = pl.program_id(0)
    fo_ref[...] = f_ref[0] + f_ref[1]

    @pl.when(i == 0)
    def _():
        en_ref[...] = (-ae_ref[...] * 0.5 * jnp.sum(sq_ref[...])
                       + jnp.sum(rep_ref[...]))


def _tc_final(f_part, rep_part, sq_part, ae):
    return pl.pallas_call(
        _tc_final_body,
        grid=(TC_GRID,),
        in_specs=[pl.BlockSpec((NC, 3, RB, 128), lambda i: (0, 0, i, 0)),
                  pl.BlockSpec((NW, 16), lambda i: (0, 0)),
                  pl.BlockSpec((NW, 16), lambda i: (0, 0)),
                  pl.BlockSpec((1, 1), lambda i: (0, 0))],
        out_specs=(pl.BlockSpec((3, RB, 128), lambda i: (0, i, 0)),
                   pl.BlockSpec((1, 1), lambda i: (0, 0))),
        out_shape=(jax.ShapeDtypeStruct((3, 784, 128), jnp.float32),
                   jax.ShapeDtypeStruct((1, 1), jnp.float32)),
    )(f_part, rep_part, sq_part, ae)


def kernel(r, amp_d, ls_d, amp_r, ls_r, amp_e, src, dst):
    sp = lambda v: jnp.logaddexp(v, 0.0)
    a_d = sp(amp_d)
    l_d = sp(ls_d)
    a_r = sp(amp_r)
    l_r = sp(ls_r)
    a_e = sp(amp_e)
    params = jnp.zeros((16,), jnp.float32)
    params = params.at[0].set(a_d).at[1].set(l_d).at[2].set(a_r)
    params = params.at[3].set(l_r).at[4].set(a_e * l_d * 0.5)
    zeros = jnp.zeros((N_PAD,), jnp.float32)

    rt = r.T
    rx = rt[0]
    ry = rt[1]
    rz = rt[2]
    rho_part = _sc_density(rx, ry, rz, dst, params, zeros)
    f_part, rep_part, sq_part = _sc_forces(rx, ry, rz, src, dst,
                                           rho_part, params, zeros)
    fsum, en = _tc_final(f_part.reshape(NC, 3, 784, 128), rep_part, sq_part,
                         a_e.astype(jnp.float32).reshape(1, 1))

    forces = fsum.reshape(3, N_PAD)[:, :N_NODES].T
    return en[0, 0], forces


# speedup vs baseline: 1.2435x; 1.2435x over previous
import functools

import jax
import jax.numpy as jnp
from jax import lax
from jax.experimental import pallas as pl
from jax.experimental.pallas import tpu as pltpu
from jax.experimental.pallas import tpu_sc as plsc

N_NODES = 100000
N_EDGES = 3200000
NC = 2
NS = 16
NW = NC * NS
E_PER_W = N_EDGES // NW
CHUNK = 4000
N_CHUNKS = E_PER_W // CHUNK
GROUPS = CHUNK // 16
N_PAD = 784 * 128
NPS = N_PAD // NS
W_GROUPS = NPS // 16
RB = 112
TC_GRID = 784 // RB

_mesh = plsc.VectorSubcoreMesh(core_axis_name="c", subcore_axis_name="s")
_sc_params = pltpu.CompilerParams(needs_layout_passes=False)


def _rsqrt16(b2, iters=2):
    i = plsc.bitcast(b2, jnp.int32)
    i = jnp.int32(0x5F3759DF) - (i >> 1)
    y = plsc.bitcast(i, jnp.float32)
    for _ in range(iters):
        y = y * (1.5 - 0.5 * b2 * y * y)
    return y


def _edge_geom(rx_buf, ry_buf, rz_buf, gi):
    gsl = pl.ds(gi * 16, 16)
    x = rx_buf[gsl]
    y = ry_buf[gsl]
    z = rz_buf[gsl]
    b2 = jnp.maximum(x * x + y * y + z * z, 1e-30)
    rinv = _rsqrt16(b2)
    return x, y, z, b2 * rinv, rinv


@functools.partial(
    pl.kernel,
    out_type=jax.ShapeDtypeStruct((NC, N_PAD), jnp.float32),
    mesh=_mesh,
    compiler_params=_sc_params,
    scratch_types=[
        pltpu.VMEM_SHARED((N_PAD,), jnp.float32),
        pltpu.VMEM((16,), jnp.float32),
    ] + [pltpu.VMEM((CHUNK,), jnp.float32)] * 3
      + [pltpu.VMEM((CHUNK,), jnp.int32)] * 2
      + [pltpu.VMEM((CHUNK,), jnp.float32)] * 2
      + [pltpu.SemaphoreType.DMA] * 2,
)
def _sc_density(rx_hbm, ry_hbm, rz_hbm, dst_hbm, par_hbm, zeros_hbm, rho_out,
                rho_sh, par_v,
                rx_buf, ry_buf, rz_buf, dsta, dstb, da, db,
                ssc_a, ssc_b):
    cid = lax.axis_index("c")
    sid = lax.axis_index("s")
    wid = sid * NC + cid
    sl = pl.ds(sid * NPS, NPS)
    pltpu.sync_copy(zeros_hbm.at[sl], rho_sh.at[sl])
    pltpu.sync_copy(par_hbm, par_v)
    plsc.subcore_barrier()

    pv = par_v[...]
    a_d = pv[0]
    l_d = pv[1]
    e_base = wid * E_PER_W

    def do_chunk(e0, dst_buf, d_buf, sem):
        pltpu.sync_copy(rx_hbm.at[pl.ds(e0, CHUNK)], rx_buf)
        pltpu.sync_copy(ry_hbm.at[pl.ds(e0, CHUNK)], ry_buf)
        pltpu.sync_copy(rz_hbm.at[pl.ds(e0, CHUNK)], rz_buf)
        pltpu.sync_copy(dst_hbm.at[pl.ds(e0, CHUNK)], dst_buf)

        @plsc.parallel_loop(0, GROUPS, unroll=2)
        def _(gi):
            _, _, _, b, _ = _edge_geom(rx_buf, ry_buf, rz_buf, gi)
            d_buf[pl.ds(gi * 16, 16)] = a_d * jnp.exp(-l_d * b)

        return pltpu.async_copy(d_buf, rho_sh.at[dst_buf], sem, add=True)

    def pair_body(ci, carry):
        e0 = e_base + ci * (2 * CHUNK)
        sc_a = do_chunk(e0, dsta, da, ssc_a)
        sc_b = do_chunk(e0 + CHUNK, dstb, db, ssc_b)
        sc_a.wait()
        sc_b.wait()
        return carry

    lax.fori_loop(0, (N_CHUNKS - 1) // 2, pair_body, 0)
    do_chunk(e_base + (N_CHUNKS - 1) * CHUNK, dsta, da, ssc_a).wait()
    plsc.subcore_barrier()
    pltpu.sync_copy(rho_sh.at[sl], rho_out.at[cid, sl])


@functools.partial(
    pl.kernel,
    out_type=(jax.ShapeDtypeStruct((NC * 3, N_PAD), jnp.float32),
              jax.ShapeDtypeStruct((NW, 16), jnp.float32),
              jax.ShapeDtypeStruct((NW, 16), jnp.float32)),
    mesh=_mesh,
    compiler_params=_sc_params,
    scratch_types=[
        pltpu.VMEM_SHARED((N_PAD,), jnp.float32),
        pltpu.VMEM_SHARED((N_PAD,), jnp.float32),
        pltpu.VMEM_SHARED((N_PAD,), jnp.float32),
        pltpu.VMEM_SHARED((N_PAD,), jnp.float32),
        pltpu.VMEM((NPS,), jnp.float32),
        pltpu.VMEM((NPS,), jnp.float32),
        pltpu.VMEM((16,), jnp.float32),
        pltpu.VMEM((16,), jnp.float32),
        pltpu.VMEM((16,), jnp.float32),
    ] + [pltpu.VMEM((CHUNK,), jnp.float32)] * 3
      + [pltpu.VMEM((CHUNK,), jnp.int32)] * 2
      + [pltpu.VMEM((CHUNK,), jnp.float32)] * 7
      + [pltpu.SemaphoreType.DMA],
)
def _sc_forces(rx_hbm, ry_hbm, rz_hbm, src_hbm, dst_hbm, rho_hbm, par_hbm,
               zeros_hbm, f_out, rep_out, sq_out,
               w_sh, fx_sh, fy_sh, fz_sh, p0_v, p1_v,
               par_v, acc_buf, sq_buf,
               rx_buf, ry_buf, rz_buf, src_buf, dst_buf,
               wv_buf, pfx, pfy, pfz, nfx, nfy, nfz,
               sem):
    cid = lax.axis_index("c")
    sid = lax.axis_index("s")
    wid = sid * NC + cid
    sl = pl.ds(sid * NPS, NPS)
    pltpu.sync_copy(rho_hbm.at[0, sl], p0_v)
    pltpu.sync_copy(rho_hbm.at[1, sl], p1_v)
    pltpu.sync_copy(zeros_hbm.at[sl], fx_sh.at[sl])
    pltpu.sync_copy(zeros_hbm.at[sl], fy_sh.at[sl])
    pltpu.sync_copy(zeros_hbm.at[sl], fz_sh.at[sl])
    pltpu.sync_copy(par_hbm, par_v)

    def wgrp(gi, acc):
        gsl = pl.ds(gi * 16, 16)
        rho = p0_v[gsl] + p1_v[gsl]
        wv = _rsqrt16(rho, iters=3)
        p0_v[gsl] = wv
        return acc + rho * wv

    sq = lax.fori_loop(0, W_GROUPS, wgrp, jnp.zeros((16,), jnp.float32))
    sq_buf[...] = sq
    pltpu.sync_copy(sq_buf, sq_out.at[wid])
    pltpu.sync_copy(p0_v, w_sh.at[sl])
    plsc.subcore_barrier()

    pv = par_v[...]
    a_d = pv[0]
    l_d = pv[1]
    a_r = pv[2]
    l_r = pv[3]
    k1 = pv[4]
    e_base = wid * E_PER_W

    def chunk_body(ci, acc):
        e0 = e_base + ci * CHUNK
        pltpu.sync_copy(rx_hbm.at[pl.ds(e0, CHUNK)], rx_buf)
        pltpu.sync_copy(ry_hbm.at[pl.ds(e0, CHUNK)], ry_buf)
        pltpu.sync_copy(rz_hbm.at[pl.ds(e0, CHUNK)], rz_buf)
        pltpu.sync_copy(src_hbm.at[pl.ds(e0, CHUNK)], src_buf)
        pltpu.sync_copy(dst_hbm.at[pl.ds(e0, CHUNK)], dst_buf)
        pltpu.async_copy(w_sh.at[dst_buf], wv_buf, sem).wait()

        def grp(gi, acc):
            x, y, z, b, rinv = _edge_geom(rx_buf, ry_buf, rz_buf, gi)
            gsl = pl.ds(gi * 16, 16)
            wv = wv_buf[gsl]
            d = a_d * jnp.exp(-l_d * b)
            rep = a_r * jnp.exp(-l_r * b)
            s = (l_r * rep - k1 * d * wv) * rinv
            sx = s * x
            sy = s * y
            sz = s * z
            pfx[gsl] = sx
            pfy[gsl] = sy
            pfz[gsl] = sz
            nfx[gsl] = -sx
            nfy[gsl] = -sy
            nfz[gsl] = -sz
            return acc + rep

        acc = lax.fori_loop(0, GROUPS, grp, acc)
        pltpu.sync_copy(pfx, fx_sh.at[dst_buf], add=True)
        pltpu.sync_copy(pfy, fy_sh.at[dst_buf], add=True)
        pltpu.sync_copy(pfz, fz_sh.at[dst_buf], add=True)
        pltpu.sync_copy(nfx, fx_sh.at[src_buf], add=True)
        pltpu.sync_copy(nfy, fy_sh.at[src_buf], add=True)
        pltpu.sync_copy(nfz, fz_sh.at[src_buf], add=True)
        return acc

    acc = lax.fori_loop(0, N_CHUNKS, chunk_body, jnp.zeros((16,), jnp.float32))
    acc_buf[...] = acc
    pltpu.sync_copy(acc_buf, rep_out.at[wid])
    plsc.subcore_barrier()
    pltpu.sync_copy(fx_sh.at[sl], f_out.at[cid * 3, sl])
    pltpu.sync_copy(fy_sh.at[sl], f_out.at[cid * 3 + 1, sl])
    pltpu.sync_copy(fz_sh.at[sl], f_out.at[cid * 3 + 2, sl])


def _tc_final_body(f_ref, rep_ref, sq_ref, ae_ref, fo_ref, en_ref):
    i = pl.program_id(0)
    fo_ref[...] = f_ref[0] + f_ref[1]

    @pl.when(i == 0)
    def _():
        en_ref[...] = (-ae_ref[...] * 0.5 * jnp.sum(sq_ref[...])
                       + jnp.sum(rep_ref[...]))


def _tc_final(f_part, rep_part, sq_part, ae):
    return pl.pallas_call(
        _tc_final_body,
        grid=(TC_GRID,),
        in_specs=[pl.BlockSpec((NC, 3, RB, 128), lambda i: (0, 0, i, 0)),
                  pl.BlockSpec((NW, 16), lambda i: (0, 0)),
                  pl.BlockSpec((NW, 16), lambda i: (0, 0)),
                  pl.BlockSpec((1, 1), lambda i: (0, 0))],
        out_specs=(pl.BlockSpec((3, RB, 128), lambda i: (0, i, 0)),
                   pl.BlockSpec((1, 1), lambda i: (0, 0))),
        out_shape=(jax.ShapeDtypeStruct((3, 784, 128), jnp.float32),
                   jax.ShapeDtypeStruct((1, 1), jnp.float32)),
    )(f_part, rep_part, sq_part, ae)


def kernel(r, amp_d, ls_d, amp_r, ls_r, amp_e, src, dst):
    sp = lambda v: jnp.logaddexp(v, 0.0)
    a_d = sp(amp_d)
    l_d = sp(ls_d)
    a_r = sp(amp_r)
    l_r = sp(ls_r)
    a_e = sp(amp_e)
    params = jnp.zeros((16,), jnp.float32)
    params = params.at[0].set(a_d).at[1].set(l_d).at[2].set(a_r)
    params = params.at[3].set(l_r).at[4].set(a_e * l_d * 0.5)
    zeros = jnp.zeros((N_PAD,), jnp.float32)

    rt = r.T
    rx = rt[0]
    ry = rt[1]
    rz = rt[2]
    rho_part = _sc_density(rx, ry, rz, dst, params, zeros)
    f_part, rep_part, sq_part = _sc_forces(rx, ry, rz, src, dst,
                                           rho_part, params, zeros)
    fsum, en = _tc_final(f_part.reshape(NC, 3, 784, 128), rep_part, sq_part,
                         a_e.astype(jnp.float32).reshape(1, 1))

    forces = fsum.reshape(3, N_PAD)[:, :N_NODES].T
    return en[0, 0], forces


# speedup vs baseline: 1.4130x; 1.1363x over previous
import functools

import jax
import jax.numpy as jnp
from jax import lax
from jax.experimental import pallas as pl
from jax.experimental.pallas import tpu as pltpu
from jax.experimental.pallas import tpu_sc as plsc

N_NODES = 100000
N_EDGES = 3200000
NC = 2
NS = 16
NW = NC * NS
E_PER_W = N_EDGES // NW
CHUNK = 4000
N_CHUNKS = E_PER_W // CHUNK
GROUPS = CHUNK // 16
N_PAD = 784 * 128
NPS = N_PAD // NS
W_GROUPS = NPS // 16
RB = 112
TC_GRID = 784 // RB

_mesh = plsc.VectorSubcoreMesh(core_axis_name="c", subcore_axis_name="s")
_sc_params = pltpu.CompilerParams(needs_layout_passes=False)


def _rsqrt16(b2, iters=2):
    i = plsc.bitcast(b2, jnp.int32)
    i = jnp.int32(0x5F3759DF) - (i >> 1)
    y = plsc.bitcast(i, jnp.float32)
    for _ in range(iters):
        y = y * (1.5 - 0.5 * b2 * y * y)
    return y


def _edge_geom(rx_buf, ry_buf, rz_buf, gi):
    gsl = pl.ds(gi * 16, 16)
    x = rx_buf[gsl]
    y = ry_buf[gsl]
    z = rz_buf[gsl]
    b2 = jnp.maximum(x * x + y * y + z * z, 1e-30)
    rinv = _rsqrt16(b2)
    return x, y, z, b2 * rinv, rinv


@functools.partial(
    pl.kernel,
    out_type=jax.ShapeDtypeStruct((NC, N_PAD), jnp.float32),
    mesh=_mesh,
    compiler_params=_sc_params,
    scratch_types=[
        pltpu.VMEM_SHARED((N_PAD,), jnp.float32),
        pltpu.VMEM((16,), jnp.float32),
    ] + [pltpu.VMEM((CHUNK,), jnp.float32)] * 3
      + [pltpu.VMEM((CHUNK,), jnp.int32)] * 2
      + [pltpu.VMEM((CHUNK,), jnp.float32)] * 2
      + [pltpu.SemaphoreType.DMA] * 2,
)
def _sc_density(rx_hbm, ry_hbm, rz_hbm, dst_hbm, par_hbm, zeros_hbm, rho_out,
                rho_sh, par_v,
                rx_buf, ry_buf, rz_buf, dsta, dstb, da, db,
                ssc_a, ssc_b):
    cid = lax.axis_index("c")
    sid = lax.axis_index("s")
    wid = sid * NC + cid
    sl = pl.ds(sid * NPS, NPS)
    pltpu.sync_copy(zeros_hbm.at[sl], rho_sh.at[sl])
    pltpu.sync_copy(par_hbm, par_v)
    plsc.subcore_barrier()

    pv = par_v[...]
    a_d = pv[0]
    l_d = pv[1]
    e_base = wid * E_PER_W

    def do_chunk(e0, dst_buf, d_buf, sem):
        esl = pl.ds(e0, CHUNK)
        ins = [pltpu.async_copy(rx_hbm.at[esl], rx_buf, sem),
               pltpu.async_copy(ry_hbm.at[esl], ry_buf, sem),
               pltpu.async_copy(rz_hbm.at[esl], rz_buf, sem),
               pltpu.async_copy(dst_hbm.at[esl], dst_buf, sem)]
        for c in ins:
            c.wait()

        @plsc.parallel_loop(0, GROUPS, unroll=2)
        def _(gi):
            _, _, _, b, _ = _edge_geom(rx_buf, ry_buf, rz_buf, gi)
            d_buf[pl.ds(gi * 16, 16)] = a_d * jnp.exp(-l_d * b)

        return pltpu.async_copy(d_buf, rho_sh.at[dst_buf], sem, add=True)

    def pair_body(ci, carry):
        e0 = e_base + ci * (2 * CHUNK)
        sc_a = do_chunk(e0, dsta, da, ssc_a)
        sc_b = do_chunk(e0 + CHUNK, dstb, db, ssc_b)
        sc_a.wait()
        sc_b.wait()
        return carry

    lax.fori_loop(0, (N_CHUNKS - 1) // 2, pair_body, 0)
    do_chunk(e_base + (N_CHUNKS - 1) * CHUNK, dsta, da, ssc_a).wait()
    plsc.subcore_barrier()
    pltpu.sync_copy(rho_sh.at[sl], rho_out.at[cid, sl])


@functools.partial(
    pl.kernel,
    out_type=(jax.ShapeDtypeStruct((NC * 3, N_PAD), jnp.float32),
              jax.ShapeDtypeStruct((NW, 16), jnp.float32),
              jax.ShapeDtypeStruct((NW, 16), jnp.float32)),
    mesh=_mesh,
    compiler_params=_sc_params,
    scratch_types=[
        pltpu.VMEM_SHARED((N_PAD,), jnp.float32),
        pltpu.VMEM_SHARED((N_PAD,), jnp.float32),
        pltpu.VMEM_SHARED((N_PAD,), jnp.float32),
        pltpu.VMEM_SHARED((N_PAD,), jnp.float32),
        pltpu.VMEM((NPS,), jnp.float32),
        pltpu.VMEM((NPS,), jnp.float32),
        pltpu.VMEM((16,), jnp.float32),
        pltpu.VMEM((16,), jnp.float32),
        pltpu.VMEM((16,), jnp.float32),
    ] + [pltpu.VMEM((CHUNK,), jnp.float32)] * 3
      + [pltpu.VMEM((CHUNK,), jnp.int32)] * 2
      + [pltpu.VMEM((CHUNK,), jnp.float32)] * 7
      + [pltpu.SemaphoreType.DMA] * 2,
)
def _sc_forces(rx_hbm, ry_hbm, rz_hbm, src_hbm, dst_hbm, rho_hbm, par_hbm,
               zeros_hbm, f_out, rep_out, sq_out,
               w_sh, fx_sh, fy_sh, fz_sh, p0_v, p1_v,
               par_v, acc_buf, sq_buf,
               rx_buf, ry_buf, rz_buf, src_buf, dst_buf,
               wv_buf, pfx, pfy, pfz, nfx, nfy, nfz,
               sem, sem_dst):
    cid = lax.axis_index("c")
    sid = lax.axis_index("s")
    wid = sid * NC + cid
    sl = pl.ds(sid * NPS, NPS)
    pltpu.sync_copy(rho_hbm.at[0, sl], p0_v)
    pltpu.sync_copy(rho_hbm.at[1, sl], p1_v)
    pltpu.sync_copy(zeros_hbm.at[sl], fx_sh.at[sl])
    pltpu.sync_copy(zeros_hbm.at[sl], fy_sh.at[sl])
    pltpu.sync_copy(zeros_hbm.at[sl], fz_sh.at[sl])
    pltpu.sync_copy(par_hbm, par_v)

    def wgrp(gi, acc):
        gsl = pl.ds(gi * 16, 16)
        rho = p0_v[gsl] + p1_v[gsl]
        wv = _rsqrt16(rho, iters=3)
        p0_v[gsl] = wv
        return acc + rho * wv

    sq = lax.fori_loop(0, W_GROUPS, wgrp, jnp.zeros((16,), jnp.float32))
    sq_buf[...] = sq
    pltpu.sync_copy(sq_buf, sq_out.at[wid])
    pltpu.sync_copy(p0_v, w_sh.at[sl])
    plsc.subcore_barrier()

    pv = par_v[...]
    a_d = pv[0]
    l_d = pv[1]
    a_r = pv[2]
    l_r = pv[3]
    k1 = pv[4]
    e_base = wid * E_PER_W

    def chunk_body(ci, acc):
        e0 = e_base + ci * CHUNK
        esl = pl.ds(e0, CHUNK)
        in_dst = pltpu.async_copy(dst_hbm.at[esl], dst_buf, sem_dst)
        ins = [pltpu.async_copy(rx_hbm.at[esl], rx_buf, sem),
               pltpu.async_copy(ry_hbm.at[esl], ry_buf, sem),
               pltpu.async_copy(rz_hbm.at[esl], rz_buf, sem),
               pltpu.async_copy(src_hbm.at[esl], src_buf, sem)]
        in_dst.wait()
        gw = pltpu.async_copy(w_sh.at[dst_buf], wv_buf, sem)
        for c in ins:
            c.wait()
        gw.wait()

        def grp(gi, acc):
            x, y, z, b, rinv = _edge_geom(rx_buf, ry_buf, rz_buf, gi)
            gsl = pl.ds(gi * 16, 16)
            wv = wv_buf[gsl]
            d = a_d * jnp.exp(-l_d * b)
            rep = a_r * jnp.exp(-l_r * b)
            s = (l_r * rep - k1 * d * wv) * rinv
            sx = s * x
            sy = s * y
            sz = s * z
            pfx[gsl] = sx
            pfy[gsl] = sy
            pfz[gsl] = sz
            nfx[gsl] = -sx
            nfy[gsl] = -sy
            nfz[gsl] = -sz
            return acc + rep

        acc = lax.fori_loop(0, GROUPS, grp, acc)
        pltpu.sync_copy(pfx, fx_sh.at[dst_buf], add=True)
        pltpu.sync_copy(pfy, fy_sh.at[dst_buf], add=True)
        pltpu.sync_copy(pfz, fz_sh.at[dst_buf], add=True)
        pltpu.sync_copy(nfx, fx_sh.at[src_buf], add=True)
        pltpu.sync_copy(nfy, fy_sh.at[src_buf], add=True)
        pltpu.sync_copy(nfz, fz_sh.at[src_buf], add=True)
        return acc

    acc = lax.fori_loop(0, N_CHUNKS, chunk_body, jnp.zeros((16,), jnp.float32))
    acc_buf[...] = acc
    pltpu.sync_copy(acc_buf, rep_out.at[wid])
    plsc.subcore_barrier()
    pltpu.sync_copy(fx_sh.at[sl], f_out.at[cid * 3, sl])
    pltpu.sync_copy(fy_sh.at[sl], f_out.at[cid * 3 + 1, sl])
    pltpu.sync_copy(fz_sh.at[sl], f_out.at[cid * 3 + 2, sl])


def _tc_final_body(f_ref, rep_ref, sq_ref, ae_ref, fo_ref, en_ref):
    i = pl.program_id(0)
    fo_ref[...] = f_ref[0] + f_ref[1]

    @pl.when(i == 0)
    def _():
        en_ref[...] = (-ae_ref[...] * 0.5 * jnp.sum(sq_ref[...])
                       + jnp.sum(rep_ref[...]))


def _tc_final(f_part, rep_part, sq_part, ae):
    return pl.pallas_call(
        _tc_final_body,
        grid=(TC_GRID,),
        in_specs=[pl.BlockSpec((NC, 3, RB, 128), lambda i: (0, 0, i, 0)),
                  pl.BlockSpec((NW, 16), lambda i: (0, 0)),
                  pl.BlockSpec((NW, 16), lambda i: (0, 0)),
                  pl.BlockSpec((1, 1), lambda i: (0, 0))],
        out_specs=(pl.BlockSpec((3, RB, 128), lambda i: (0, i, 0)),
                   pl.BlockSpec((1, 1), lambda i: (0, 0))),
        out_shape=(jax.ShapeDtypeStruct((3, 784, 128), jnp.float32),
                   jax.ShapeDtypeStruct((1, 1), jnp.float32)),
    )(f_part, rep_part, sq_part, ae)


def kernel(r, amp_d, ls_d, amp_r, ls_r, amp_e, src, dst):
    sp = lambda v: jnp.logaddexp(v, 0.0)
    a_d = sp(amp_d)
    l_d = sp(ls_d)
    a_r = sp(amp_r)
    l_r = sp(ls_r)
    a_e = sp(amp_e)
    params = jnp.zeros((16,), jnp.float32)
    params = params.at[0].set(a_d).at[1].set(l_d).at[2].set(a_r)
    params = params.at[3].set(l_r).at[4].set(a_e * l_d * 0.5)
    zeros = jnp.zeros((N_PAD,), jnp.float32)

    rt = r.T
    rx = rt[0]
    ry = rt[1]
    rz = rt[2]
    rho_part = _sc_density(rx, ry, rz, dst, params, zeros)
    f_part, rep_part, sq_part = _sc_forces(rx, ry, rz, src, dst,
                                           rho_part, params, zeros)
    fsum, en = _tc_final(f_part.reshape(NC, 3, 784, 128), rep_part, sq_part,
                         a_e.astype(jnp.float32).reshape(1, 1))

    forces = fsum.reshape(3, N_PAD)[:, :N_NODES].T
    return en[0, 0], forces


# speedup vs baseline: 1.4349x; 1.0155x over previous
import functools

import jax
import jax.numpy as jnp
from jax import lax
from jax.experimental import pallas as pl
from jax.experimental.pallas import tpu as pltpu
from jax.experimental.pallas import tpu_sc as plsc

N_NODES = 100000
N_EDGES = 3200000
NC = 2
NS = 16
NW = NC * NS
E_PER_W = N_EDGES // NW
CHUNK = 4000
N_CHUNKS = E_PER_W // CHUNK
GROUPS = CHUNK // 16
N_PAD = 784 * 128
NPS = N_PAD // NS
W_GROUPS = NPS // 16
RB = 112
TC_GRID = 784 // RB

_mesh = plsc.VectorSubcoreMesh(core_axis_name="c", subcore_axis_name="s")
_sc_params = pltpu.CompilerParams(needs_layout_passes=False)


def _rsqrt16(b2, iters=2):
    i = plsc.bitcast(b2, jnp.int32)
    i = jnp.int32(0x5F3759DF) - (i >> 1)
    y = plsc.bitcast(i, jnp.float32)
    for _ in range(iters):
        y = y * (1.5 - 0.5 * b2 * y * y)
    return y


def _edge_geom(rx_buf, ry_buf, rz_buf, gi):
    gsl = pl.ds(gi * 16, 16)
    x = rx_buf[gsl]
    y = ry_buf[gsl]
    z = rz_buf[gsl]
    b2 = jnp.maximum(x * x + y * y + z * z, 1e-30)
    rinv = _rsqrt16(b2)
    return x, y, z, b2 * rinv, rinv


@functools.partial(
    pl.kernel,
    out_type=jax.ShapeDtypeStruct((NC, N_PAD), jnp.float32),
    mesh=_mesh,
    compiler_params=_sc_params,
    scratch_types=[
        pltpu.VMEM_SHARED((N_PAD,), jnp.float32),
        pltpu.VMEM((16,), jnp.float32),
    ] + ([pltpu.VMEM((CHUNK,), jnp.float32)] * 3
         + [pltpu.VMEM((CHUNK,), jnp.int32)]
         + [pltpu.VMEM((CHUNK,), jnp.float32)]) * 2
      + [pltpu.SemaphoreType.DMA] * 4,
)
def _sc_density(rx_hbm, ry_hbm, rz_hbm, dst_hbm, par_hbm, zeros_hbm, rho_out,
                rho_sh, par_v,
                rxa, rya, rza, dsta, da,
                rxb, ryb, rzb, dstb, db,
                sin_a, sin_b, ssc_a, ssc_b):
    cid = lax.axis_index("c")
    sid = lax.axis_index("s")
    wid = sid * NC + cid
    sl = pl.ds(sid * NPS, NPS)
    pltpu.sync_copy(zeros_hbm.at[sl], rho_sh.at[sl])
    pltpu.sync_copy(par_hbm, par_v)
    plsc.subcore_barrier()

    pv = par_v[...]
    a_d = pv[0]
    l_d = pv[1]
    e_base = wid * E_PER_W

    bufs_a = (rxa, rya, rza, dsta, da, sin_a, ssc_a)
    bufs_b = (rxb, ryb, rzb, dstb, db, sin_b, ssc_b)

    def start_in(e0, bufs):
        esl = pl.ds(e0, CHUNK)
        pltpu.async_copy(rx_hbm.at[esl], bufs[0], bufs[5])
        pltpu.async_copy(ry_hbm.at[esl], bufs[1], bufs[5])
        pltpu.async_copy(rz_hbm.at[esl], bufs[2], bufs[5])
        pltpu.async_copy(dst_hbm.at[esl], bufs[3], bufs[5])

    def drain_in(bufs):
        esl = pl.ds(0, CHUNK)
        pltpu.make_async_copy(rx_hbm.at[esl], bufs[0], bufs[5]).wait()
        pltpu.make_async_copy(ry_hbm.at[esl], bufs[1], bufs[5]).wait()
        pltpu.make_async_copy(rz_hbm.at[esl], bufs[2], bufs[5]).wait()
        pltpu.make_async_copy(dst_hbm.at[esl], bufs[3], bufs[5]).wait()

    def compute(bufs):
        @plsc.parallel_loop(0, GROUPS, unroll=2)
        def _(gi):
            _, _, _, b, _ = _edge_geom(bufs[0], bufs[1], bufs[2], gi)
            bufs[4][pl.ds(gi * 16, 16)] = a_d * jnp.exp(-l_d * b)

        return pltpu.async_copy(bufs[4], rho_sh.at[bufs[3]], bufs[6],
                                add=True)

    def pair_body(ci, carry):
        e0 = e_base + ci * (2 * CHUNK)
        start_in(e0, bufs_a)
        start_in(e0 + CHUNK, bufs_b)
        drain_in(bufs_a)
        sc_a = compute(bufs_a)
        drain_in(bufs_b)
        sc_b = compute(bufs_b)
        sc_a.wait()
        sc_b.wait()
        return carry

    lax.fori_loop(0, (N_CHUNKS - 1) // 2, pair_body, 0)
    start_in(e_base + (N_CHUNKS - 1) * CHUNK, bufs_a)
    drain_in(bufs_a)
    compute(bufs_a).wait()
    plsc.subcore_barrier()
    pltpu.sync_copy(rho_sh.at[sl], rho_out.at[cid, sl])


@functools.partial(
    pl.kernel,
    out_type=(jax.ShapeDtypeStruct((NC * 3, N_PAD), jnp.float32),
              jax.ShapeDtypeStruct((NW, 16), jnp.float32),
              jax.ShapeDtypeStruct((NW, 16), jnp.float32)),
    mesh=_mesh,
    compiler_params=_sc_params,
    scratch_types=[
        pltpu.VMEM_SHARED((N_PAD,), jnp.float32),
        pltpu.VMEM_SHARED((N_PAD,), jnp.float32),
        pltpu.VMEM_SHARED((N_PAD,), jnp.float32),
        pltpu.VMEM_SHARED((N_PAD,), jnp.float32),
        pltpu.VMEM((NPS,), jnp.float32),
        pltpu.VMEM((NPS,), jnp.float32),
        pltpu.VMEM((16,), jnp.float32),
        pltpu.VMEM((16,), jnp.float32),
        pltpu.VMEM((16,), jnp.float32),
    ] + [pltpu.VMEM((CHUNK,), jnp.float32)] * 3
      + [pltpu.VMEM((CHUNK,), jnp.int32)] * 2
      + [pltpu.VMEM((CHUNK,), jnp.float32)] * 7
      + [pltpu.SemaphoreType.DMA] * 2,
)
def _sc_forces(rx_hbm, ry_hbm, rz_hbm, src_hbm, dst_hbm, rho_hbm, par_hbm,
               zeros_hbm, f_out, rep_out, sq_out,
               w_sh, fx_sh, fy_sh, fz_sh, p0_v, p1_v,
               par_v, acc_buf, sq_buf,
               rx_buf, ry_buf, rz_buf, src_buf, dst_buf,
               wv_buf, pfx, pfy, pfz, nfx, nfy, nfz,
               sem, sem_dst):
    cid = lax.axis_index("c")
    sid = lax.axis_index("s")
    wid = sid * NC + cid
    sl = pl.ds(sid * NPS, NPS)
    pltpu.sync_copy(rho_hbm.at[0, sl], p0_v)
    pltpu.sync_copy(rho_hbm.at[1, sl], p1_v)
    pltpu.sync_copy(zeros_hbm.at[sl], fx_sh.at[sl])
    pltpu.sync_copy(zeros_hbm.at[sl], fy_sh.at[sl])
    pltpu.sync_copy(zeros_hbm.at[sl], fz_sh.at[sl])
    pltpu.sync_copy(par_hbm, par_v)

    def wgrp(gi, acc):
        gsl = pl.ds(gi * 16, 16)
        rho = p0_v[gsl] + p1_v[gsl]
        wv = _rsqrt16(rho, iters=3)
        p0_v[gsl] = wv
        return acc + rho * wv

    sq = lax.fori_loop(0, W_GROUPS, wgrp, jnp.zeros((16,), jnp.float32))
    sq_buf[...] = sq
    pltpu.sync_copy(sq_buf, sq_out.at[wid])
    pltpu.sync_copy(p0_v, w_sh.at[sl])
    plsc.subcore_barrier()

    pv = par_v[...]
    a_d = pv[0]
    l_d = pv[1]
    a_r = pv[2]
    l_r = pv[3]
    k1 = pv[4]
    e_base = wid * E_PER_W

    def chunk_body(ci, acc):
        e0 = e_base + ci * CHUNK
        esl = pl.ds(e0, CHUNK)
        in_dst = pltpu.async_copy(dst_hbm.at[esl], dst_buf, sem_dst)
        ins = [pltpu.async_copy(rx_hbm.at[esl], rx_buf, sem),
               pltpu.async_copy(ry_hbm.at[esl], ry_buf, sem),
               pltpu.async_copy(rz_hbm.at[esl], rz_buf, sem),
               pltpu.async_copy(src_hbm.at[esl], src_buf, sem)]
        in_dst.wait()
        gw = pltpu.async_copy(w_sh.at[dst_buf], wv_buf, sem)
        for c in ins:
            c.wait()
        gw.wait()

        def grp(gi, acc):
            x, y, z, b, rinv = _edge_geom(rx_buf, ry_buf, rz_buf, gi)
            gsl = pl.ds(gi * 16, 16)
            wv = wv_buf[gsl]
            d = a_d * jnp.exp(-l_d * b)
            rep = a_r * jnp.exp(-l_r * b)
            s = (l_r * rep - k1 * d * wv) * rinv
            sx = s * x
            sy = s * y
            sz = s * z
            pfx[gsl] = sx
            pfy[gsl] = sy
            pfz[gsl] = sz
            nfx[gsl] = -sx
            nfy[gsl] = -sy
            nfz[gsl] = -sz
            return acc + rep

        acc = lax.fori_loop(0, GROUPS, grp, acc)
        pltpu.sync_copy(pfx, fx_sh.at[dst_buf], add=True)
        pltpu.sync_copy(pfy, fy_sh.at[dst_buf], add=True)
        pltpu.sync_copy(pfz, fz_sh.at[dst_buf], add=True)
        pltpu.sync_copy(nfx, fx_sh.at[src_buf], add=True)
        pltpu.sync_copy(nfy, fy_sh.at[src_buf], add=True)
        pltpu.sync_copy(nfz, fz_sh.at[src_buf], add=True)
        return acc

    acc = lax.fori_loop(0, N_CHUNKS, chunk_body, jnp.zeros((16,), jnp.float32))
    acc_buf[...] = acc
    pltpu.sync_copy(acc_buf, rep_out.at[wid])
    plsc.subcore_barrier()
    pltpu.sync_copy(fx_sh.at[sl], f_out.at[cid * 3, sl])
    pltpu.sync_copy(fy_sh.at[sl], f_out.at[cid * 3 + 1, sl])
    pltpu.sync_copy(fz_sh.at[sl], f_out.at[cid * 3 + 2, sl])


def _tc_final_body(f_ref, rep_ref, sq_ref, ae_ref, fo_ref, en_ref):
    i = pl.program_id(0)
    fo_ref[...] = f_ref[0] + f_ref[1]

    @pl.when(i == 0)
    def _():
        en_ref[...] = (-ae_ref[...] * 0.5 * jnp.sum(sq_ref[...])
                       + jnp.sum(rep_ref[...]))


def _tc_final(f_part, rep_part, sq_part, ae):
    return pl.pallas_call(
        _tc_final_body,
        grid=(TC_GRID,),
        in_specs=[pl.BlockSpec((NC, 3, RB, 128), lambda i: (0, 0, i, 0)),
                  pl.BlockSpec((NW, 16), lambda i: (0, 0)),
                  pl.BlockSpec((NW, 16), lambda i: (0, 0)),
                  pl.BlockSpec((1, 1), lambda i: (0, 0))],
        out_specs=(pl.BlockSpec((3, RB, 128), lambda i: (0, i, 0)),
                   pl.BlockSpec((1, 1), lambda i: (0, 0))),
        out_shape=(jax.ShapeDtypeStruct((3, 784, 128), jnp.float32),
                   jax.ShapeDtypeStruct((1, 1), jnp.float32)),
    )(f_part, rep_part, sq_part, ae)


def kernel(r, amp_d, ls_d, amp_r, ls_r, amp_e, src, dst):
    sp = lambda v: jnp.logaddexp(v, 0.0)
    a_d = sp(amp_d)
    l_d = sp(ls_d)
    a_r = sp(amp_r)
    l_r = sp(ls_r)
    a_e = sp(amp_e)
    params = jnp.zeros((16,), jnp.float32)
    params = params.at[0].set(a_d).at[1].set(l_d).at[2].set(a_r)
    params = params.at[3].set(l_r).at[4].set(a_e * l_d * 0.5)
    zeros = jnp.zeros((N_PAD,), jnp.float32)

    rt = r.T
    rx = rt[0]
    ry = rt[1]
    rz = rt[2]
    rho_part = _sc_density(rx, ry, rz, dst, params, zeros)
    f_part, rep_part, sq_part = _sc_forces(rx, ry, rz, src, dst,
                                           rho_part, params, zeros)
    fsum, en = _tc_final(f_part.reshape(NC, 3, 784, 128), rep_part, sq_part,
                         a_e.astype(jnp.float32).reshape(1, 1))

    forces = fsum.reshape(3, N_PAD)[:, :N_NODES].T
    return en[0, 0], forces
